# baseline (speedup 1.0000x reference)
import functools

import jax
import jax.numpy as jnp
from jax import lax
from jax.experimental import pallas as pl
from jax.experimental.pallas import tpu as pltpu
from jax.experimental.pallas import tpu_sc as plsc

FT_DIM = 64
N_ACTIVE = 32
_C = 1
_IDX_PER_CHUNK = _C * N_ACTIVE
_NBUF = 8
_LANES = 16
_D_REGS = FT_DIM // _LANES


@functools.lru_cache(maxsize=None)
def _make_sc_gather_sum(B):
    info = plsc.get_sparse_core_info()
    num_workers = info.num_cores * info.num_subcores
    bpw = B // num_workers
    nchunk = bpw // _C
    assert bpw % _C == 0 and nchunk % 2 == 0
    mesh = plsc.VectorSubcoreMesh(core_axis_name="c", subcore_axis_name="s")

    @functools.partial(
        pl.kernel,
        mesh=mesh,
        out_type=jax.ShapeDtypeStruct((B, FT_DIM), jnp.float32),
        scratch_types=[
            pltpu.VMEM((bpw, N_ACTIVE), jnp.int32),
        ] + [pltpu.VMEM((_IDX_PER_CHUNK, FT_DIM), jnp.float32)] * _NBUF + [
            pltpu.VMEM((bpw, FT_DIM), jnp.float32),
        ] + [pltpu.SemaphoreType.DMA] * _NBUF,
        compiler_params=pltpu.CompilerParams(use_tc_tiling_on_sc=False),
    )
    def sc_gather_sum(idx_hbm, ftw_hbm, out_hbm, idx_v, *rest):
        rows_bufs = rest[:_NBUF]
        acc_v = rest[_NBUF]
        sems = rest[_NBUF + 1:]
        wid = lax.axis_index("s") * info.num_cores + lax.axis_index("c")
        ibase = pl.multiple_of(wid * bpw, 8)
        pltpu.sync_copy(idx_hbm.at[pl.ds(ibase, bpw)], idx_v)

        def start_gather(chunk, rows, sem):
            pltpu.make_async_copy(
                ftw_hbm.at[idx_v.at[chunk]],
                rows, sem).start()

        def wait_gather(rows, sem):
            pltpu.make_async_copy(
                ftw_hbm.at[idx_v.at[0]],
                rows, sem).wait()

        def accum(chunk, rows):
            zero = jnp.zeros((_LANES,), jnp.float32)
            unroll = 4

            def jbody(j2, accs):
                out = list(accs)
                for ju in range(unroll):
                    for d in range(_D_REGS):
                        out[d] = (out[d] +
                                  rows[unroll * j2 + ju,
                                       pl.ds(d * _LANES, _LANES)])
                return tuple(out)

            accs = lax.fori_loop(0, N_ACTIVE // unroll, jbody,
                                 (zero,) * _D_REGS)
            for d in range(_D_REGS):
                acc_v[chunk, pl.ds(d * _LANES, _LANES)] = accs[d]

        bufs = tuple(zip(rows_bufs, sems))
        nbuf = len(bufs)
        for b, (rows, sem) in enumerate(bufs):
            start_gather(b, rows, sem)

        def kbody(k, carry):
            g = nbuf * k
            for b, (rows, sem) in enumerate(bufs):
                wait_gather(rows, sem)
                accum(g + b, rows)
                start_gather(g + b + nbuf, rows, sem)
            return carry

        lax.fori_loop(0, nchunk // nbuf - 1, kbody, 0)
        for b, (rows, sem) in enumerate(bufs):
            wait_gather(rows, sem)
            accum(nchunk - nbuf + b, rows)
        pltpu.sync_copy(acc_v, out_hbm.at[pl.ds(wid * bpw, bpw)])

    return sc_gather_sum


def _tc_mlp_body(acc_ref, dense_ref, ftb_ref, w1_ref, b1_ref,
                 w2_ref, b2_ref, w3_ref, b3_ref, out_ref):
    matmul_t = functools.partial(
        lax.dot_general,
        dimension_numbers=(((1,), (1,)), ((), ())),
        preferred_element_type=jnp.float32)
    w1 = w1_ref[...]
    w1s = w1[:, :FT_DIM] + w1[:, FT_DIM:2 * FT_DIM]
    w1d = w1[:, 2 * FT_DIM:]
    a = jnp.maximum(acc_ref[...] + ftb_ref[...], 0.0)
    h1 = matmul_t(a, w1s) + matmul_t(dense_ref[...], w1d)
    h1 = jnp.maximum(h1 + b1_ref[...], 0.0)
    h2 = jnp.maximum(matmul_t(h1, w2_ref[...]) + b2_ref[...], 0.0)
    h2t = h2.T
    y = lax.dot_general(w3_ref[...], h2t, (((1,), (0,)), ((), ())),
                        preferred_element_type=jnp.float32)
    out_ref[...] = jnp.tanh(y + b3_ref[...])


def _tc_mlp(acc, dense, ftb, w1, b1, w2, b2, w3, b3):
    B = acc.shape[0]
    BT = 2048
    rep = lambda shape: pl.BlockSpec(shape, lambda i: (0, 0))
    out2d = pl.pallas_call(
        _tc_mlp_body,
        grid=(B // BT,),
        in_specs=[
            pl.BlockSpec((BT, FT_DIM), lambda i: (i, 0)),
            pl.BlockSpec((BT, 16), lambda i: (i, 0)),
            rep((1, 64)),
            rep((64, 144)),
            rep((1, 64)),
            rep((32, 64)),
            rep((1, 32)),
            rep((1, 32)),
            rep((1, 1)),
        ],
        out_specs=pl.BlockSpec((1, BT), lambda i: (0, i)),
        out_shape=jax.ShapeDtypeStruct((1, B), jnp.float32),
    )(acc, dense, ftb, w1, b1, w2, b2, w3, b3)
    return out2d.reshape(B)


def kernel(sparse_batch, dense_batch, ft_w, ft_b,
           fc1_w, fc1_b, fc2_w, fc2_b, fc3_w, fc3_b):
    B = sparse_batch.shape[0]
    acc = _make_sc_gather_sum(B)(sparse_batch, ft_w)
    return _tc_mlp(acc, dense_batch, ft_b.reshape(1, FT_DIM),
                   fc1_w, fc1_b.reshape(1, -1),
                   fc2_w, fc2_b.reshape(1, -1),
                   fc3_w, fc3_b.reshape(1, 1))


# speedup vs baseline: 1.0632x; 1.0632x over previous
import functools

import jax
import jax.numpy as jnp
from jax import lax
from jax.experimental import pallas as pl
from jax.experimental.pallas import tpu as pltpu
from jax.experimental.pallas import tpu_sc as plsc

FT_DIM = 64
N_ACTIVE = 32
_IDX_PAD = 128
_C = 4
_IDX_PER_CHUNK = _C * N_ACTIVE
_NBUF = 8
_LANES = 16
_D_REGS = FT_DIM // _LANES


@functools.lru_cache(maxsize=None)
def _make_sc_gather_sum(B):
    info = plsc.get_sparse_core_info()
    num_workers = info.num_cores * info.num_subcores
    bpw = B // num_workers
    nchunk = bpw // _C
    assert bpw % _C == 0 and nchunk % _NBUF == 0
    mesh = plsc.VectorSubcoreMesh(core_axis_name="c", subcore_axis_name="s")

    @functools.partial(
        pl.kernel,
        mesh=mesh,
        out_type=jax.ShapeDtypeStruct((B, FT_DIM), jnp.float32),
        scratch_types=[
            pltpu.VMEM((bpw // 2, N_ACTIVE), jnp.int32),
            pltpu.VMEM((bpw * N_ACTIVE,), jnp.int32),
        ] + [pltpu.VMEM((_IDX_PER_CHUNK, FT_DIM), jnp.float32)] * _NBUF + [
            pltpu.VMEM((bpw, FT_DIM), jnp.float32),
        ] + [pltpu.SemaphoreType.DMA] * _NBUF,
        compiler_params=pltpu.CompilerParams(use_tc_tiling_on_sc=False),
    )
    def sc_gather_sum(idx_hbm, ftw_hbm, out_hbm, idx2d_v, idx_v, *rest):
        rows_bufs = rest[:_NBUF]
        acc_v = rest[_NBUF]
        sems = rest[_NBUF + 1:]
        wid = lax.axis_index("s") * info.num_cores + lax.axis_index("c")
        ibase = pl.multiple_of(wid * bpw, 8)
        half_rows = bpw // 2
        for half in range(2):
            pltpu.sync_copy(
                idx_hbm.at[pl.ds(ibase + half * half_rows, half_rows),
                           pl.ds(0, N_ACTIVE)],
                idx2d_v)
            fbase = half * half_rows * N_ACTIVE

            def cbody(e2, carry):
                for eu in range(2):
                    e = 2 * e2 + eu
                    for d in range(N_ACTIVE // _LANES):
                        idx_v[pl.ds(fbase + e * N_ACTIVE + d * _LANES,
                                    _LANES)] = \
                            idx2d_v[e, pl.ds(d * _LANES, _LANES)]
                return carry

            lax.fori_loop(0, half_rows // 2, cbody, 0)

        def start_gather(chunk, rows, sem):
            off = pl.multiple_of(chunk * _IDX_PER_CHUNK, 8)
            pltpu.make_async_copy(
                ftw_hbm.at[idx_v.at[pl.ds(off, _IDX_PER_CHUNK)]],
                rows, sem).start()

        def wait_gather(rows, sem):
            pltpu.make_async_copy(
                ftw_hbm.at[idx_v.at[pl.ds(0, _IDX_PER_CHUNK)]],
                rows, sem).wait()

        def accum(chunk, rows):
            rbase = chunk * _C
            zero = jnp.zeros((_LANES,), jnp.float32)

            def jbody(j2, accs):
                out = list(accs)
                for ju in range(2):
                    for e in range(_C):
                        for d in range(_D_REGS):
                            out[e * _D_REGS + d] = (
                                out[e * _D_REGS + d] +
                                rows[e * N_ACTIVE + 2 * j2 + ju,
                                     pl.ds(d * _LANES, _LANES)])
                return tuple(out)

            accs = lax.fori_loop(0, N_ACTIVE // 2, jbody,
                                 (zero,) * (_C * _D_REGS))
            for e in range(_C):
                for d in range(_D_REGS):
                    acc_v[rbase + e, pl.ds(d * _LANES, _LANES)] = \
                        accs[e * _D_REGS + d]

        bufs = tuple(zip(rows_bufs, sems))
        nbuf = len(bufs)
        for b, (rows, sem) in enumerate(bufs):
            start_gather(b, rows, sem)

        def kbody(k, carry):
            g = nbuf * k
            for b, (rows, sem) in enumerate(bufs):
                wait_gather(rows, sem)
                accum(g + b, rows)
                start_gather(g + b + nbuf, rows, sem)
            return carry

        lax.fori_loop(0, nchunk // nbuf - 1, kbody, 0)
        for b, (rows, sem) in enumerate(bufs):
            wait_gather(rows, sem)
            accum(nchunk - nbuf + b, rows)
        pltpu.sync_copy(acc_v, out_hbm.at[pl.ds(wid * bpw, bpw)])

    return sc_gather_sum


def _tc_mlp_body(acc_ref, dense_ref, ftb_ref, w1_ref, b1_ref,
                 w2_ref, b2_ref, w3_ref, b3_ref, out_ref):
    matmul_t = functools.partial(
        lax.dot_general,
        dimension_numbers=(((1,), (1,)), ((), ())),
        preferred_element_type=jnp.float32)
    w1 = w1_ref[...]
    w1s = w1[:, :FT_DIM] + w1[:, FT_DIM:2 * FT_DIM]
    w1d = w1[:, 2 * FT_DIM:]
    a = jnp.maximum(acc_ref[...] + ftb_ref[...], 0.0)
    h1 = matmul_t(a, w1s) + matmul_t(dense_ref[...], w1d)
    h1 = jnp.maximum(h1 + b1_ref[...], 0.0)
    h2 = jnp.maximum(matmul_t(h1, w2_ref[...]) + b2_ref[...], 0.0)
    h2t = h2.T
    y = lax.dot_general(w3_ref[...], h2t, (((1,), (0,)), ((), ())),
                        preferred_element_type=jnp.float32)
    y2 = jnp.tanh(y + b3_ref[...])
    out_ref[...] = y2.reshape(out_ref.shape)


def _tc_mlp(acc, dense, ftb, w1, b1, w2, b2, w3, b3):
    B = acc.shape[0]
    BT = 2048
    rep = lambda shape: pl.BlockSpec(shape, lambda i: (0, 0))
    out2d = pl.pallas_call(
        _tc_mlp_body,
        grid=(B // BT,),
        in_specs=[
            pl.BlockSpec((BT, FT_DIM), lambda i: (i, 0)),
            pl.BlockSpec((BT, 16), lambda i: (i, 0)),
            rep((1, 64)),
            rep((64, 144)),
            rep((1, 64)),
            rep((32, 64)),
            rep((1, 32)),
            rep((1, 32)),
            rep((1, 1)),
        ],
        out_specs=pl.BlockSpec((BT // 128, 128), lambda i: (i, 0)),
        out_shape=jax.ShapeDtypeStruct((B // 128, 128), jnp.float32),
    )(acc, dense, ftb, w1, b1, w2, b2, w3, b3)
    return out2d.reshape(B)


def kernel(sparse_batch, dense_batch, ft_w, ft_b,
           fc1_w, fc1_b, fc2_w, fc2_b, fc3_w, fc3_b):
    B = sparse_batch.shape[0]
    idx_pad = jnp.pad(sparse_batch, ((0, 0), (0, _IDX_PAD - N_ACTIVE)))
    acc = _make_sc_gather_sum(B)(idx_pad, ft_w)
    return _tc_mlp(acc, dense_batch, ft_b.reshape(1, FT_DIM),
                   fc1_w, fc1_b.reshape(1, -1),
                   fc2_w, fc2_b.reshape(1, -1),
                   fc3_w, fc3_b.reshape(1, 1))


# speedup vs baseline: 1.0662x; 1.0028x over previous
import functools

import jax
import jax.numpy as jnp
from jax import lax
from jax.experimental import pallas as pl
from jax.experimental.pallas import tpu as pltpu
from jax.experimental.pallas import tpu_sc as plsc

FT_DIM = 64
N_ACTIVE = 32
_IDX_PAD = 128
_C = 8
_IDX_PER_XFER = 128
_XFERS = _C * N_ACTIVE // _IDX_PER_XFER
_NBUF = 4
_LANES = 16
_D_REGS = FT_DIM // _LANES


@functools.lru_cache(maxsize=None)
def _make_sc_gather_sum(B):
    info = plsc.get_sparse_core_info()
    num_workers = info.num_cores * info.num_subcores
    bpw = B // num_workers
    nchunk = bpw // _C
    assert bpw % _C == 0 and nchunk % _NBUF == 0, (bpw, nchunk)
    mesh = plsc.VectorSubcoreMesh(core_axis_name="c", subcore_axis_name="s")

    @functools.partial(
        pl.kernel,
        mesh=mesh,
        out_type=jax.ShapeDtypeStruct((B, FT_DIM), jnp.float32),
        scratch_types=[
            pltpu.VMEM((bpw // 2, N_ACTIVE), jnp.int32),
            pltpu.VMEM((bpw * N_ACTIVE,), jnp.int32),
        ] + [pltpu.VMEM((_C * N_ACTIVE, FT_DIM), jnp.float32)] * _NBUF + [
            pltpu.VMEM((bpw, FT_DIM), jnp.float32),
        ] + [pltpu.SemaphoreType.DMA] * _NBUF,
        compiler_params=pltpu.CompilerParams(use_tc_tiling_on_sc=False),
    )
    def sc_gather_sum(idx_hbm, ftw_hbm, out_hbm, idx2d_v, idx_v, *rest):
        rows_bufs = rest[:_NBUF]
        acc_v = rest[_NBUF]
        sems = rest[_NBUF + 1:]
        wid = lax.axis_index("s") * info.num_cores + lax.axis_index("c")
        ibase = pl.multiple_of(wid * bpw, 8)
        half_rows = bpw // 2
        for half in range(2):
            pltpu.sync_copy(
                idx_hbm.at[pl.ds(ibase + half * half_rows, half_rows),
                           pl.ds(0, N_ACTIVE)],
                idx2d_v)
            fbase = half * half_rows * N_ACTIVE

            def cbody(e2, carry):
                for eu in range(2):
                    e = 2 * e2 + eu
                    for d in range(N_ACTIVE // _LANES):
                        idx_v[pl.ds(fbase + e * N_ACTIVE + d * _LANES,
                                    _LANES)] = \
                            idx2d_v[e, pl.ds(d * _LANES, _LANES)]
                return carry

            lax.fori_loop(0, half_rows // 2, cbody, 0)

        def start_gather(chunk, rows, sem):
            for t in range(_XFERS):
                off = pl.multiple_of(
                    chunk * (_C * N_ACTIVE) + t * _IDX_PER_XFER, 8)
                pltpu.make_async_copy(
                    ftw_hbm.at[idx_v.at[pl.ds(off, _IDX_PER_XFER)]],
                    rows.at[pl.ds(t * _IDX_PER_XFER, _IDX_PER_XFER)],
                    sem).start()

        def wait_gather(rows, sem):
            pltpu.make_async_copy(
                ftw_hbm.at[idx_v.at[pl.ds(0, _IDX_PER_XFER)]],
                rows, sem).wait()

        def accum(chunk, rows):
            rbase = chunk * _C
            zero = jnp.zeros((_LANES,), jnp.float32)
            for g in range(_C // 4):
                def jbody(j2, accs, g=g):
                    out = list(accs)
                    for ju in range(2):
                        for e in range(4):
                            for d in range(_D_REGS):
                                out[e * _D_REGS + d] = (
                                    out[e * _D_REGS + d] +
                                    rows[(g * 4 + e) * N_ACTIVE
                                         + 2 * j2 + ju,
                                         pl.ds(d * _LANES, _LANES)])
                    return tuple(out)

                accs = lax.fori_loop(0, N_ACTIVE // 2, jbody,
                                     (zero,) * (4 * _D_REGS))
                for e in range(4):
                    for d in range(_D_REGS):
                        acc_v[rbase + g * 4 + e,
                              pl.ds(d * _LANES, _LANES)] = \
                            accs[e * _D_REGS + d]

        bufs = tuple(zip(rows_bufs, sems))
        nbuf = len(bufs)
        for b, (rows, sem) in enumerate(bufs):
            start_gather(b, rows, sem)

        def kbody(k, carry):
            g = nbuf * k
            for b, (rows, sem) in enumerate(bufs):
                wait_gather(rows, sem)
                accum(g + b, rows)
                start_gather(g + b + nbuf, rows, sem)
            return carry

        lax.fori_loop(0, nchunk // nbuf - 1, kbody, 0)
        for b, (rows, sem) in enumerate(bufs):
            wait_gather(rows, sem)
            accum(nchunk - nbuf + b, rows)
        pltpu.sync_copy(acc_v, out_hbm.at[pl.ds(wid * bpw, bpw)])

    return sc_gather_sum


def _tc_mlp_body(acc_ref, dense_ref, ftb_ref, w1_ref, b1_ref,
                 w2_ref, b2_ref, w3_ref, b3_ref, out_ref):
    matmul_t = functools.partial(
        lax.dot_general,
        dimension_numbers=(((1,), (1,)), ((), ())),
        preferred_element_type=jnp.float32)
    w1 = w1_ref[...]
    w1s = w1[:, :FT_DIM] + w1[:, FT_DIM:2 * FT_DIM]
    w1d = w1[:, 2 * FT_DIM:]
    a = jnp.maximum(acc_ref[...] + ftb_ref[...], 0.0)
    h1 = matmul_t(a, w1s) + matmul_t(dense_ref[...], w1d)
    h1 = jnp.maximum(h1 + b1_ref[...], 0.0)
    h2 = jnp.maximum(matmul_t(h1, w2_ref[...]) + b2_ref[...], 0.0)
    h2t = h2.T
    y = lax.dot_general(w3_ref[...], h2t, (((1,), (0,)), ((), ())),
                        preferred_element_type=jnp.float32)
    y2 = jnp.tanh(y + b3_ref[...])
    out_ref[...] = y2.reshape(out_ref.shape)


def _tc_mlp(acc, dense, ftb, w1, b1, w2, b2, w3, b3):
    B = acc.shape[0]
    BT = 2048
    rep = lambda shape: pl.BlockSpec(shape, lambda i: (0, 0))
    out2d = pl.pallas_call(
        _tc_mlp_body,
        grid=(B // BT,),
        in_specs=[
            pl.BlockSpec((BT, FT_DIM), lambda i: (i, 0)),
            pl.BlockSpec((BT, 16), lambda i: (i, 0)),
            rep((1, 64)),
            rep((64, 144)),
            rep((1, 64)),
            rep((32, 64)),
            rep((1, 32)),
            rep((1, 32)),
            rep((1, 1)),
        ],
        out_specs=pl.BlockSpec((BT // 128, 128), lambda i: (i, 0)),
        out_shape=jax.ShapeDtypeStruct((B // 128, 128), jnp.float32),
    )(acc, dense, ftb, w1, b1, w2, b2, w3, b3)
    return out2d.reshape(B)


def kernel(sparse_batch, dense_batch, ft_w, ft_b,
           fc1_w, fc1_b, fc2_w, fc2_b, fc3_w, fc3_b):
    B = sparse_batch.shape[0]
    idx_pad = jnp.pad(sparse_batch, ((0, 0), (0, _IDX_PAD - N_ACTIVE)))
    acc = _make_sc_gather_sum(B)(idx_pad, ft_w)
    return _tc_mlp(acc, dense_batch, ft_b.reshape(1, FT_DIM),
                   fc1_w, fc1_b.reshape(1, -1),
                   fc2_w, fc2_b.reshape(1, -1),
                   fc3_w, fc3_b.reshape(1, 1))
